# Initial kernel scaffold; baseline (speedup 1.0000x reference)
import jax
import jax.numpy as jnp
from jax.experimental import pallas as pl


def kernel(data, length, embed, W, b):
    raise NotImplementedError("write your pallas kernel here")


# speedup vs baseline: 2.5227x; 2.5227x over previous
import functools

import jax
import jax.numpy as jnp
from jax import lax
from jax.experimental import pallas as pl
from jax.experimental.pallas import tpu as pltpu
from jax.experimental.pallas import tpu_sc as plsc

VOCAB = 1000000
EMB = 32
OUT = 5
B = 4096
L = 200

NC = 2
NS = 16
NW = NC * NS
BAGS_PER_W = B // NW
CHUNK_BAGS = 4
TOK_PER_CHUNK = CHUNK_BAGS * L
STREAM_W = 100
STREAMS_PER_CHUNK = TOK_PER_CHUNK // STREAM_W
CHUNKS = BAGS_PER_W // CHUNK_BAGS
ROWS_PER_W = (BAGS_PER_W * L) // STREAM_W


def _sc_pool(embed, data2d):
    mesh = plsc.VectorSubcoreMesh(core_axis_name="c", subcore_axis_name="s")

    @functools.partial(
        pl.kernel,
        mesh=mesh,
        out_type=jax.ShapeDtypeStruct((B, EMB), jnp.float32),
        compiler_params=pltpu.CompilerParams(use_tc_tiling_on_sc=False),
        scratch_types=[
            pltpu.VMEM((ROWS_PER_W, STREAM_W), jnp.int32),
            pltpu.VMEM((TOK_PER_CHUNK, EMB), jnp.float32),
            pltpu.VMEM((TOK_PER_CHUNK, EMB), jnp.float32),
            pltpu.VMEM((CHUNK_BAGS, EMB), jnp.float32),
            pltpu.SemaphoreType.DMA,
            pltpu.SemaphoreType.DMA,
            pltpu.SemaphoreType.DMA,
        ],
    )
    def pool(table_hbm, data_hbm, out_hbm, idx_v, rows_a, rows_b, stage_v,
             isem, sem_a, sem_b):
        wid = lax.axis_index("s") * NC + lax.axis_index("c")
        row0 = wid * ROWS_PER_W

        pltpu.async_copy(data_hbm.at[pl.ds(row0, ROWS_PER_W)], idx_v, isem).wait()

        def fire(c, rows_v, sem):
            for j in range(STREAMS_PER_CHUNK):
                pltpu.async_copy(
                    table_hbm.at[idx_v.at[c * STREAMS_PER_CHUNK + j]],
                    rows_v.at[pl.ds(j * STREAM_W, STREAM_W)],
                    sem,
                )

        def drain(rows_v, sem):
            pltpu.make_async_copy(
                table_hbm.at[pl.ds(0, TOK_PER_CHUNK)], rows_v, sem
            ).wait()

        def acc_out(c, rows_v):
            def bag_body(i, carry2):
                zero = jnp.zeros((16,), jnp.float32)

                def row_body(r, accs):
                    a0, a1 = accs
                    base = i * L + r * 8
                    for u in range(8):
                        a0 = a0 + rows_v[base + u, 0:16]
                        a1 = a1 + rows_v[base + u, 16:32]
                    return (a0, a1)

                a0, a1 = lax.fori_loop(0, L // 8, row_body, (zero, zero))
                stage_v[i, 0:16] = a0
                stage_v[i, 16:32] = a1
                return carry2

            lax.fori_loop(0, CHUNK_BAGS, bag_body, 0)
            pltpu.sync_copy(
                stage_v,
                out_hbm.at[pl.ds(wid * BAGS_PER_W + c * CHUNK_BAGS, CHUNK_BAGS)],
            )

        fire(0, rows_a, sem_a)

        def pair_body(k, carry):
            c0 = k * 2
            fire(c0 + 1, rows_b, sem_b)
            drain(rows_a, sem_a)
            acc_out(c0, rows_a)

            @pl.when(k < CHUNKS // 2 - 1)
            def _():
                fire(c0 + 2, rows_a, sem_a)

            drain(rows_b, sem_b)
            acc_out(c0 + 1, rows_b)
            return carry

        lax.fori_loop(0, CHUNKS // 2, pair_body, 0)

    return pool(embed, data2d)


def _tc_head(pooled, inv_len, wt, b2):
    BLK = 512

    def body(p_ref, il_ref, w_ref, b_ref, o_ref):
        x = p_ref[:] * il_ref[:]
        y = jnp.dot(x, w_ref[:], preferred_element_type=jnp.float32)
        o_ref[:] = y + b_ref[:]

    return pl.pallas_call(
        body,
        grid=(B // BLK,),
        in_specs=[
            pl.BlockSpec((BLK, EMB), lambda i: (i, 0)),
            pl.BlockSpec((BLK, 1), lambda i: (i, 0)),
            pl.BlockSpec((EMB, OUT), lambda i: (0, 0)),
            pl.BlockSpec((1, OUT), lambda i: (0, 0)),
        ],
        out_specs=pl.BlockSpec((BLK, OUT), lambda i: (i, 0)),
        out_shape=jax.ShapeDtypeStruct((B, OUT), jnp.float32),
    )(pooled, inv_len, wt, b2)


def kernel(data, length, embed, W, b):
    data2d = data.reshape(B * L // STREAM_W, STREAM_W)
    pooled = _sc_pool(embed, data2d)
    inv_len = (1.0 / length.astype(jnp.float32)).reshape(B, 1)
    return _tc_head(pooled, inv_len, W.T, b.reshape(1, OUT))
